# baseline (speedup 1.0000x reference)
import functools

import jax
import jax.numpy as jnp
from jax import lax
from jax.experimental import pallas as pl
from jax.experimental.pallas import tpu as pltpu
from jax.experimental.pallas import tpu_sc as plsc

VOCAB = 1000000
EMBED = 64
HID = 128
B = 4096
T = 50

_NC = 2
_NS = 16
_NW = _NC * _NS
_N = B * T
_PER_W = _N // _NW
_CHUNK = 128
_NCH = _PER_W // _CHUNK


@functools.cache
def _make_gather():
    mesh = plsc.VectorSubcoreMesh(core_axis_name="c", subcore_axis_name="s")
    return pl.kernel(
        _gather_body,
        mesh=mesh,
        out_type=jax.ShapeDtypeStruct((_N, 2 * EMBED), jnp.float32),
        scratch_types=[
            pltpu.VMEM((_PER_W,), jnp.int32),
            pltpu.VMEM((2, _CHUNK, 2 * EMBED), jnp.float32),
            pltpu.SemaphoreType.DMA,
            pltpu.SemaphoreType.DMA,
        ],
    )


def _gather_body(table_hbm, idx_hbm, out_hbm, idx_v, rows_v, sem0, sem1):
    wid = lax.axis_index("s") * _NC + lax.axis_index("c")
    base = wid * _PER_W
    pltpu.sync_copy(idx_hbm.at[pl.ds(base, _PER_W)], idx_v)

    def _idx(j):
        return idx_v.at[pl.ds(j * _CHUNK, _CHUNK)]

    pltpu.async_copy(table_hbm.at[_idx(0)], rows_v.at[0], sem0)

    def pair_body(p, carry):
        j0 = p * 2
        j1 = j0 + 1
        pltpu.async_copy(table_hbm.at[_idx(j1)], rows_v.at[1], sem1)
        pltpu.make_async_copy(
            table_hbm.at[_idx(j0)], rows_v.at[0], sem0).wait()
        pltpu.sync_copy(
            rows_v.at[0], out_hbm.at[pl.ds(base + j0 * _CHUNK, _CHUNK)])

        @pl.when(j0 + 2 < _NCH)
        def _():
            pltpu.async_copy(table_hbm.at[_idx(j0 + 2)], rows_v.at[0], sem0)

        pltpu.make_async_copy(
            table_hbm.at[_idx(j1)], rows_v.at[1], sem1).wait()
        pltpu.sync_copy(
            rows_v.at[1], out_hbm.at[pl.ds(base + j1 * _CHUNK, _CHUNK)])
        return carry

    lax.fori_loop(0, _NCH // 2, pair_body, 0)


_PBLK = 2048


def _pad_body(tT_ref, eye_ref, out_ref):
    x = tT_ref[...]
    xT = jax.lax.dot_general(
        x, eye_ref[...], (((0,), (0,)), ((), ())),
        preferred_element_type=jnp.float32)
    out_ref[:, :EMBED] = xT


def _pad_table(tableT):
    return pl.pallas_call(
        _pad_body,
        grid=((VOCAB + _PBLK - 1) // _PBLK,),
        in_specs=[
            pl.BlockSpec((EMBED, _PBLK), lambda i: (0, i)),
            pl.BlockSpec((EMBED, EMBED), lambda i: (0, 0)),
        ],
        out_specs=pl.BlockSpec((_PBLK, 2 * EMBED), lambda i: (i, 0)),
        out_shape=jax.ShapeDtypeStruct((VOCAB, 2 * EMBED), jnp.float32),
        compiler_params=pltpu.CompilerParams(
            dimension_semantics=("arbitrary",)),
    )(tableT, jnp.eye(EMBED, dtype=jnp.float32))


_BB = 1024
_NB = B // _BB
_TG = 8
_NG = (T + _TG - 1) // _TG


def _lstm_body(*refs):
    emb_refs = refs[:_TG]
    wx_ref, wh_ref, bi_ref, bh_ref, out_ref, h_ref, c_ref = refs[_TG:]
    g = pl.program_id(1)

    @pl.when(g == 0)
    def _():
        h_ref[...] = jnp.zeros_like(h_ref)
        c_ref[...] = jnp.zeros_like(c_ref)

    bias = bi_ref[...] + bh_ref[...]

    def step(k):
        xt = emb_refs[k][:, :EMBED]
        h = h_ref[...]
        gates = (
            jnp.dot(xt, wx_ref[...], preferred_element_type=jnp.float32)
            + jnp.dot(h, wh_ref[...], preferred_element_type=jnp.float32)
            + bias
        )
        i = jax.nn.sigmoid(gates[:, 0 * HID:1 * HID])
        f = jax.nn.sigmoid(gates[:, 1 * HID:2 * HID])
        gg = jnp.tanh(gates[:, 2 * HID:3 * HID])
        o = jax.nn.sigmoid(gates[:, 3 * HID:4 * HID])
        c = f * c_ref[...] + i * gg
        h = o * jnp.tanh(c)
        c_ref[...] = c
        h_ref[...] = h
        out_ref[k, :, :] = h

    for k in range(_TG):
        if (_NG - 1) * _TG + k < T:
            step(k)
        else:
            pl.when(g < _NG - 1)(lambda k=k: step(k))


def _emb_spec(k):
    return pl.BlockSpec(
        (_BB, 2 * EMBED),
        lambda b, g, k=k: (jnp.minimum(g * _TG + k, T - 1) * _NB + b, 0))


def _lstm(emb, wx, wh, bi, bh, *, interpret=False):
    return pl.pallas_call(
        _lstm_body,
        grid=(_NB, _NG),
        in_specs=[_emb_spec(k) for k in range(_TG)] + [
            pl.BlockSpec((EMBED, 4 * HID), lambda b, g: (0, 0)),
            pl.BlockSpec((HID, 4 * HID), lambda b, g: (0, 0)),
            pl.BlockSpec((1, 4 * HID), lambda b, g: (0, 0)),
            pl.BlockSpec((1, 4 * HID), lambda b, g: (0, 0)),
        ],
        out_specs=[
            pl.BlockSpec((_TG, _BB, HID), lambda b, g: (g, b, 0)),
            pl.BlockSpec((_BB, HID), lambda b, g: (b, 0)),
            pl.BlockSpec((_BB, HID), lambda b, g: (b, 0)),
        ],
        out_shape=[
            jax.ShapeDtypeStruct((T, B, HID), jnp.float32),
            jax.ShapeDtypeStruct((B, HID), jnp.float32),
            jax.ShapeDtypeStruct((B, HID), jnp.float32),
        ],
        compiler_params=pltpu.CompilerParams(
            dimension_semantics=("arbitrary", "arbitrary")),
        interpret=interpret,
    )(*([emb] * _TG), wx, wh, bi, bh)


def kernel(x, table, W_ih, W_hh, b_ih, b_hh):
    idx = x.astype(jnp.int32).T.reshape(_N)
    table128 = _pad_table(table.T)
    emb = _make_gather()(table128, idx)
    wx = W_ih.T
    wh = W_hh.T
    out3, hT, cT = _lstm(emb, wx, wh,
                         b_ih.reshape(1, -1), b_hh.reshape(1, -1))
    return jnp.transpose(out3, (1, 0, 2)), hT, cT


# speedup vs baseline: 1.4449x; 1.4449x over previous
import functools

import jax
import jax.numpy as jnp
from jax import lax
from jax.experimental import pallas as pl
from jax.experimental.pallas import tpu as pltpu
from jax.experimental.pallas import tpu_sc as plsc

VOCAB = 1000000
EMBED = 64
HID = 128
B = 4096
T = 50

_NC = 2
_NS = 16
_NW = _NC * _NS
_N = B * T
_PER_W = _N // _NW
_CHUNK = 128
_NCH = _PER_W // _CHUNK


@functools.cache
def _make_gather():
    mesh = plsc.VectorSubcoreMesh(core_axis_name="c", subcore_axis_name="s")
    return pl.kernel(
        _gather_body,
        mesh=mesh,
        out_type=jax.ShapeDtypeStruct((_N, 2 * EMBED), jnp.float32),
        scratch_types=[
            pltpu.VMEM((_PER_W,), jnp.int32),
            pltpu.VMEM((2, _CHUNK, 2 * EMBED), jnp.float32),
            pltpu.SemaphoreType.DMA,
            pltpu.SemaphoreType.DMA,
        ],
    )


def _gather_body(table_hbm, idx_hbm, out_hbm, idx_v, rows_v, sem0, sem1):
    wid = lax.axis_index("s") * _NC + lax.axis_index("c")
    base = wid * _PER_W
    pltpu.sync_copy(idx_hbm.at[pl.ds(base, _PER_W)], idx_v)

    def _idx(j):
        return idx_v.at[pl.ds(j * _CHUNK, _CHUNK)]

    pltpu.async_copy(table_hbm.at[_idx(0)], rows_v.at[0], sem0)

    def pair_body(p, carry):
        j0 = p * 2
        j1 = j0 + 1
        pltpu.async_copy(table_hbm.at[_idx(j1)], rows_v.at[1], sem1)
        pltpu.make_async_copy(
            table_hbm.at[_idx(j0)], rows_v.at[0], sem0).wait()
        pltpu.sync_copy(
            rows_v.at[0], out_hbm.at[pl.ds(base + j0 * _CHUNK, _CHUNK)])

        @pl.when(j0 + 2 < _NCH)
        def _():
            pltpu.async_copy(table_hbm.at[_idx(j0 + 2)], rows_v.at[0], sem0)

        pltpu.make_async_copy(
            table_hbm.at[_idx(j1)], rows_v.at[1], sem1).wait()
        pltpu.sync_copy(
            rows_v.at[1], out_hbm.at[pl.ds(base + j1 * _CHUNK, _CHUNK)])
        return carry

    lax.fori_loop(0, _NCH // 2, pair_body, 0)


_PBLK = 8192


def _pad_body(tT_ref, eye_ref, out_ref):
    x = tT_ref[...]
    xT = jax.lax.dot_general(
        x, eye_ref[...], (((0,), (0,)), ((), ())),
        preferred_element_type=jnp.float32)
    out_ref[:, :EMBED] = xT


def _pad_table(tableT):
    return pl.pallas_call(
        _pad_body,
        grid=((VOCAB + _PBLK - 1) // _PBLK,),
        in_specs=[
            pl.BlockSpec((EMBED, _PBLK), lambda i: (0, i)),
            pl.BlockSpec((EMBED, EMBED), lambda i: (0, 0)),
        ],
        out_specs=pl.BlockSpec((_PBLK, 2 * EMBED), lambda i: (i, 0)),
        out_shape=jax.ShapeDtypeStruct((VOCAB, 2 * EMBED), jnp.float32),
        compiler_params=pltpu.CompilerParams(
            dimension_semantics=("arbitrary",)),
    )(tableT, jnp.eye(EMBED, dtype=jnp.float32))


_BB = 1024
_NB = B // _BB
_TG = 8
_NG = (T + _TG - 1) // _TG


def _lstm_body(*refs):
    emb_refs = refs[:_TG]
    wx_ref, wh_ref, bi_ref, bh_ref, out_ref, h_ref, c_ref = refs[_TG:]
    g = pl.program_id(1)

    @pl.when(g == 0)
    def _():
        h_ref[...] = jnp.zeros_like(h_ref)
        c_ref[...] = jnp.zeros_like(c_ref)

    bias = bi_ref[...] + bh_ref[...]

    def step(k):
        xt = emb_refs[k][:, :EMBED]
        h = h_ref[...]
        gates = (
            jnp.dot(xt, wx_ref[...], preferred_element_type=jnp.float32)
            + jnp.dot(h, wh_ref[...], preferred_element_type=jnp.float32)
            + bias
        )
        i = jax.nn.sigmoid(gates[:, 0 * HID:1 * HID])
        f = jax.nn.sigmoid(gates[:, 1 * HID:2 * HID])
        gg = jnp.tanh(gates[:, 2 * HID:3 * HID])
        o = jax.nn.sigmoid(gates[:, 3 * HID:4 * HID])
        c = f * c_ref[...] + i * gg
        h = o * jnp.tanh(c)
        c_ref[...] = c
        h_ref[...] = h
        out_ref[k, :, :] = h

    for k in range(_TG):
        if (_NG - 1) * _TG + k < T:
            step(k)
        else:
            pl.when(g < _NG - 1)(lambda k=k: step(k))


def _emb_spec(k):
    return pl.BlockSpec(
        (_BB, 2 * EMBED),
        lambda b, g, k=k: (jnp.minimum(g * _TG + k, T - 1) * _NB + b, 0))


def _lstm(emb, wx, wh, bi, bh, *, interpret=False):
    return pl.pallas_call(
        _lstm_body,
        grid=(_NB, _NG),
        in_specs=[_emb_spec(k) for k in range(_TG)] + [
            pl.BlockSpec((EMBED, 4 * HID), lambda b, g: (0, 0)),
            pl.BlockSpec((HID, 4 * HID), lambda b, g: (0, 0)),
            pl.BlockSpec((1, 4 * HID), lambda b, g: (0, 0)),
            pl.BlockSpec((1, 4 * HID), lambda b, g: (0, 0)),
        ],
        out_specs=[
            pl.BlockSpec((_TG, _BB, HID), lambda b, g: (g, b, 0)),
            pl.BlockSpec((_BB, HID), lambda b, g: (b, 0)),
            pl.BlockSpec((_BB, HID), lambda b, g: (b, 0)),
        ],
        out_shape=[
            jax.ShapeDtypeStruct((T, B, HID), jnp.float32),
            jax.ShapeDtypeStruct((B, HID), jnp.float32),
            jax.ShapeDtypeStruct((B, HID), jnp.float32),
        ],
        compiler_params=pltpu.CompilerParams(
            dimension_semantics=("arbitrary", "arbitrary")),
        interpret=interpret,
    )(*([emb] * _TG), wx, wh, bi, bh)


def kernel(x, table, W_ih, W_hh, b_ih, b_hh):
    idx = x.astype(jnp.int32).T.reshape(_N)
    table128 = _pad_table(table.T)
    emb = _make_gather()(table128, idx)
    wx = W_ih.T
    wh = W_hh.T
    out3, hT, cT = _lstm(emb, wx, wh,
                         b_ih.reshape(1, -1), b_hh.reshape(1, -1))
    return jnp.transpose(out3, (1, 0, 2)), hT, cT


# speedup vs baseline: 1.4449x; 1.0000x over previous
import functools

import jax
import jax.numpy as jnp
from jax import lax
from jax.experimental import pallas as pl
from jax.experimental.pallas import tpu as pltpu
from jax.experimental.pallas import tpu_sc as plsc

VOCAB = 1000000
EMBED = 64
HID = 128
B = 4096
T = 50

_NC = 2
_NS = 16
_NW = _NC * _NS
_N = B * T
_PER_W = _N // _NW
_CHUNK = 128
_NCH = _PER_W // _CHUNK


@functools.cache
def _make_gather():
    mesh = plsc.VectorSubcoreMesh(core_axis_name="c", subcore_axis_name="s")
    return pl.kernel(
        _gather_body,
        mesh=mesh,
        out_type=jax.ShapeDtypeStruct((_N, 2 * EMBED), jnp.float32),
        scratch_types=[
            pltpu.VMEM((_PER_W,), jnp.int32),
            pltpu.VMEM((2, _CHUNK, 2 * EMBED), jnp.float32),
            pltpu.SemaphoreType.DMA,
            pltpu.SemaphoreType.DMA,
        ],
    )


def _gather_body(table_hbm, idx_hbm, out_hbm, idx_v, rows_v, sem0, sem1):
    wid = lax.axis_index("s") * _NC + lax.axis_index("c")
    base = wid * _PER_W
    pltpu.sync_copy(idx_hbm.at[pl.ds(base, _PER_W)], idx_v)

    def _idx(j):
        return idx_v.at[pl.ds(j * _CHUNK, _CHUNK)]

    pltpu.async_copy(table_hbm.at[_idx(0)], rows_v.at[0], sem0)

    def pair_body(p, carry):
        j0 = p * 2
        j1 = j0 + 1
        pltpu.async_copy(table_hbm.at[_idx(j1)], rows_v.at[1], sem1)
        pltpu.make_async_copy(
            table_hbm.at[_idx(j0)], rows_v.at[0], sem0).wait()
        pltpu.sync_copy(
            rows_v.at[0], out_hbm.at[pl.ds(base + j0 * _CHUNK, _CHUNK)])

        @pl.when(j0 + 2 < _NCH)
        def _():
            pltpu.async_copy(table_hbm.at[_idx(j0 + 2)], rows_v.at[0], sem0)

        pltpu.make_async_copy(
            table_hbm.at[_idx(j1)], rows_v.at[1], sem1).wait()
        pltpu.sync_copy(
            rows_v.at[1], out_hbm.at[pl.ds(base + j1 * _CHUNK, _CHUNK)])
        return carry

    lax.fori_loop(0, _NCH // 2, pair_body, 0)


_PBLK = 8192


def _pad_body(tT_ref, eye_ref, out_ref):
    x = tT_ref[...]
    xT = jax.lax.dot_general(
        x, eye_ref[...], (((0,), (0,)), ((), ())),
        preferred_element_type=jnp.float32)
    out_ref[:, :EMBED] = xT


def _pad_table(tableT):
    return pl.pallas_call(
        _pad_body,
        grid=((VOCAB + _PBLK - 1) // _PBLK,),
        in_specs=[
            pl.BlockSpec((EMBED, _PBLK), lambda i: (0, i)),
            pl.BlockSpec((EMBED, EMBED), lambda i: (0, 0)),
        ],
        out_specs=pl.BlockSpec((_PBLK, 2 * EMBED), lambda i: (i, 0)),
        out_shape=jax.ShapeDtypeStruct((VOCAB, 2 * EMBED), jnp.float32),
        compiler_params=pltpu.CompilerParams(
            dimension_semantics=("arbitrary",)),
    )(tableT, jnp.eye(EMBED, dtype=jnp.float32))


_BB = 1024
_NB = B // _BB
_TG = 10
_NG = T // _TG


def _lstm_body(*refs):
    emb_refs = refs[:_TG]
    wx_ref, wh_ref, bi_ref, bh_ref, out_ref, h_ref, c_ref = refs[_TG:]
    g = pl.program_id(1)

    @pl.when(g == 0)
    def _():
        h_ref[...] = jnp.zeros_like(h_ref)
        c_ref[...] = jnp.zeros_like(c_ref)

    bias = bi_ref[...] + bh_ref[...]

    def step(k):
        xt = emb_refs[k][:, :EMBED]
        h = h_ref[...]
        gates = (
            jnp.dot(xt, wx_ref[...], preferred_element_type=jnp.float32)
            + jnp.dot(h, wh_ref[...], preferred_element_type=jnp.float32)
            + bias
        )
        i = jax.nn.sigmoid(gates[:, 0 * HID:1 * HID])
        f = jax.nn.sigmoid(gates[:, 1 * HID:2 * HID])
        gg = jnp.tanh(gates[:, 2 * HID:3 * HID])
        o = jax.nn.sigmoid(gates[:, 3 * HID:4 * HID])
        c = f * c_ref[...] + i * gg
        h = o * jnp.tanh(c)
        c_ref[...] = c
        h_ref[...] = h
        out_ref[k, :, :] = h

    for k in range(_TG):
        step(k)


def _emb_spec(k):
    return pl.BlockSpec(
        (_BB, 2 * EMBED),
        lambda b, g, k=k: ((g * _TG + k) * _NB + b, 0))


def _lstm(emb, wx, wh, bi, bh, *, interpret=False):
    return pl.pallas_call(
        _lstm_body,
        grid=(_NB, _NG),
        in_specs=[_emb_spec(k) for k in range(_TG)] + [
            pl.BlockSpec((EMBED, 4 * HID), lambda b, g: (0, 0)),
            pl.BlockSpec((HID, 4 * HID), lambda b, g: (0, 0)),
            pl.BlockSpec((1, 4 * HID), lambda b, g: (0, 0)),
            pl.BlockSpec((1, 4 * HID), lambda b, g: (0, 0)),
        ],
        out_specs=[
            pl.BlockSpec((_TG, _BB, HID), lambda b, g: (g, b, 0)),
            pl.BlockSpec((_BB, HID), lambda b, g: (b, 0)),
            pl.BlockSpec((_BB, HID), lambda b, g: (b, 0)),
        ],
        out_shape=[
            jax.ShapeDtypeStruct((T, B, HID), jnp.float32),
            jax.ShapeDtypeStruct((B, HID), jnp.float32),
            jax.ShapeDtypeStruct((B, HID), jnp.float32),
        ],
        compiler_params=pltpu.CompilerParams(
            dimension_semantics=("arbitrary", "arbitrary")),
        interpret=interpret,
    )(*([emb] * _TG), wx, wh, bi, bh)


def kernel(x, table, W_ih, W_hh, b_ih, b_hh):
    idx = x.astype(jnp.int32).T.reshape(_N)
    table128 = _pad_table(table.T)
    emb = _make_gather()(table128, idx)
    wx = W_ih.T
    wh = W_hh.T
    out3, hT, cT = _lstm(emb, wx, wh,
                         b_ih.reshape(1, -1), b_hh.reshape(1, -1))
    return jnp.transpose(out3, (1, 0, 2)), hT, cT


# speedup vs baseline: 1.5508x; 1.0733x over previous
import functools

import jax
import jax.numpy as jnp
from jax import lax
from jax.experimental import pallas as pl
from jax.experimental.pallas import tpu as pltpu
from jax.experimental.pallas import tpu_sc as plsc

VOCAB = 1000000
EMBED = 64
HID = 128
B = 4096
T = 50

_NC = 2
_NS = 16
_NW = _NC * _NS
_N = B * T
_PER_W = _N // _NW
_CHUNK = 128
_NCH = _PER_W // _CHUNK


@functools.cache
def _make_gather():
    mesh = plsc.VectorSubcoreMesh(core_axis_name="c", subcore_axis_name="s")
    return pl.kernel(
        _gather_body,
        mesh=mesh,
        out_type=jax.ShapeDtypeStruct((_N, 2 * EMBED), jnp.float32),
        scratch_types=[
            pltpu.VMEM((_PER_W,), jnp.int32),
            pltpu.VMEM((2, _CHUNK, 2 * EMBED), jnp.float32),
            pltpu.SemaphoreType.DMA,
            pltpu.SemaphoreType.DMA,
        ],
    )


def _gather_body(table_hbm, idx_hbm, out_hbm, idx_v, rows_v, sem0, sem1):
    wid = lax.axis_index("s") * _NC + lax.axis_index("c")
    base = wid * _PER_W
    pltpu.sync_copy(idx_hbm.at[pl.ds(base, _PER_W)], idx_v)

    def _idx(j):
        return idx_v.at[pl.ds(j * _CHUNK, _CHUNK)]

    pltpu.async_copy(table_hbm.at[_idx(0)], rows_v.at[0], sem0)

    def pair_body(p, carry):
        j0 = p * 2
        j1 = j0 + 1
        pltpu.async_copy(table_hbm.at[_idx(j1)], rows_v.at[1], sem1)
        pltpu.make_async_copy(
            table_hbm.at[_idx(j0)], rows_v.at[0], sem0).wait()
        pltpu.sync_copy(
            rows_v.at[0], out_hbm.at[pl.ds(base + j0 * _CHUNK, _CHUNK)])

        @pl.when(j0 + 2 < _NCH)
        def _():
            pltpu.async_copy(table_hbm.at[_idx(j0 + 2)], rows_v.at[0], sem0)

        pltpu.make_async_copy(
            table_hbm.at[_idx(j1)], rows_v.at[1], sem1).wait()
        pltpu.sync_copy(
            rows_v.at[1], out_hbm.at[pl.ds(base + j1 * _CHUNK, _CHUNK)])
        return carry

    lax.fori_loop(0, _NCH // 2, pair_body, 0)


_PBLK = 8192


def _pad_body(tT_ref, eye_ref, out_ref):
    x = tT_ref[...]
    xT = jax.lax.dot_general(
        x, eye_ref[...], (((0,), (0,)), ((), ())),
        preferred_element_type=jnp.float32)
    out_ref[:, :EMBED] = xT
    out_ref[:, EMBED:] = jnp.zeros((_PBLK, EMBED), jnp.float32)


def _pad_table(tableT):
    return pl.pallas_call(
        _pad_body,
        grid=((VOCAB + _PBLK - 1) // _PBLK,),
        in_specs=[
            pl.BlockSpec((EMBED, _PBLK), lambda i: (0, i)),
            pl.BlockSpec((EMBED, EMBED), lambda i: (0, 0)),
        ],
        out_specs=pl.BlockSpec((_PBLK, 2 * EMBED), lambda i: (i, 0)),
        out_shape=jax.ShapeDtypeStruct((VOCAB, 2 * EMBED), jnp.float32),
        compiler_params=pltpu.CompilerParams(
            dimension_semantics=("arbitrary",)),
    )(tableT, jnp.eye(EMBED, dtype=jnp.float32))


_BB = 1024
_NB = B // _BB
_TG = 10
_NG = T // _TG


def _lstm_body(*refs):
    emb_refs = refs[:_TG]
    w_ref, bi_ref, bh_ref, out_ref, h_ref, c_ref = refs[_TG:]
    g = pl.program_id(1)

    @pl.when(g == 0)
    def _():
        h_ref[...] = jnp.zeros_like(h_ref)
        c_ref[...] = jnp.zeros_like(c_ref)

    bias = bi_ref[...] + bh_ref[...]

    def step(k):
        xh = jnp.concatenate([emb_refs[k][...], h_ref[...]], axis=1)
        gates = jnp.dot(
            xh, w_ref[...], preferred_element_type=jnp.float32) + bias
        i = jax.nn.sigmoid(gates[:, 0 * HID:1 * HID])
        f = jax.nn.sigmoid(gates[:, 1 * HID:2 * HID])
        gg = jnp.tanh(gates[:, 2 * HID:3 * HID])
        o = jax.nn.sigmoid(gates[:, 3 * HID:4 * HID])
        c = f * c_ref[...] + i * gg
        h = o * jnp.tanh(c)
        c_ref[...] = c
        h_ref[...] = h
        out_ref[k, :, :] = h

    for k in range(_TG):
        step(k)


def _emb_spec(k):
    return pl.BlockSpec(
        (_BB, 2 * EMBED),
        lambda b, g, k=k: ((g * _TG + k) * _NB + b, 0))


def _lstm(emb, w, bi, bh, *, interpret=False):
    return pl.pallas_call(
        _lstm_body,
        grid=(_NB, _NG),
        in_specs=[_emb_spec(k) for k in range(_TG)] + [
            pl.BlockSpec((2 * EMBED + HID, 4 * HID), lambda b, g: (0, 0)),
            pl.BlockSpec((1, 4 * HID), lambda b, g: (0, 0)),
            pl.BlockSpec((1, 4 * HID), lambda b, g: (0, 0)),
        ],
        out_specs=[
            pl.BlockSpec((_TG, _BB, HID), lambda b, g: (g, b, 0)),
            pl.BlockSpec((_BB, HID), lambda b, g: (b, 0)),
            pl.BlockSpec((_BB, HID), lambda b, g: (b, 0)),
        ],
        out_shape=[
            jax.ShapeDtypeStruct((T, B, HID), jnp.float32),
            jax.ShapeDtypeStruct((B, HID), jnp.float32),
            jax.ShapeDtypeStruct((B, HID), jnp.float32),
        ],
        compiler_params=pltpu.CompilerParams(
            dimension_semantics=("arbitrary", "arbitrary")),
        interpret=interpret,
    )(*([emb] * _TG), w, bi, bh)


def kernel(x, table, W_ih, W_hh, b_ih, b_hh):
    idx = x.astype(jnp.int32).T.reshape(_N)
    table128 = _pad_table(table.T)
    emb = _make_gather()(table128, idx)
    w = jnp.concatenate([
        W_ih.T,
        jnp.zeros((EMBED, 4 * HID), jnp.float32),
        W_hh.T,
    ], axis=0)
    out3, hT, cT = _lstm(emb, w,
                         b_ih.reshape(1, -1), b_hh.reshape(1, -1))
    return jnp.transpose(out3, (1, 0, 2)), hT, cT


# speedup vs baseline: 1.6326x; 1.0527x over previous
import functools

import jax
import jax.numpy as jnp
from jax import lax
from jax.experimental import pallas as pl
from jax.experimental.pallas import tpu as pltpu
from jax.experimental.pallas import tpu_sc as plsc

VOCAB = 1000000
EMBED = 64
HID = 128
B = 4096
T = 50

_NC = 2
_NS = 16
_NW = _NC * _NS
_N = B * T
_PER_W = _N // _NW
_CHUNK = 128
_NCH = _PER_W // _CHUNK


@functools.cache
def _make_gather():
    mesh = plsc.VectorSubcoreMesh(core_axis_name="c", subcore_axis_name="s")
    return pl.kernel(
        _gather_body,
        mesh=mesh,
        out_type=jax.ShapeDtypeStruct((_N, 2 * EMBED), jnp.float32),
        scratch_types=[
            pltpu.VMEM((_PER_W,), jnp.int32),
            pltpu.VMEM((2, _CHUNK, 2 * EMBED), jnp.float32),
            pltpu.SemaphoreType.DMA,
            pltpu.SemaphoreType.DMA,
        ],
    )


def _gather_body(table_hbm, idx_hbm, out_hbm, idx_v, rows_v, sem0, sem1):
    wid = lax.axis_index("s") * _NC + lax.axis_index("c")
    base = wid * _PER_W
    pltpu.sync_copy(idx_hbm.at[pl.ds(base, _PER_W)], idx_v)

    def _idx(j):
        return idx_v.at[pl.ds(j * _CHUNK, _CHUNK)]

    pltpu.async_copy(table_hbm.at[_idx(0)], rows_v.at[0], sem0)

    def pair_body(p, carry):
        j0 = p * 2
        j1 = j0 + 1
        pltpu.async_copy(table_hbm.at[_idx(j1)], rows_v.at[1], sem1)
        pltpu.make_async_copy(
            table_hbm.at[_idx(j0)], rows_v.at[0], sem0).wait()
        pltpu.sync_copy(
            rows_v.at[0], out_hbm.at[pl.ds(base + j0 * _CHUNK, _CHUNK)])

        @pl.when(j0 + 2 < _NCH)
        def _():
            pltpu.async_copy(table_hbm.at[_idx(j0 + 2)], rows_v.at[0], sem0)

        pltpu.make_async_copy(
            table_hbm.at[_idx(j1)], rows_v.at[1], sem1).wait()
        pltpu.sync_copy(
            rows_v.at[1], out_hbm.at[pl.ds(base + j1 * _CHUNK, _CHUNK)])
        return carry

    lax.fori_loop(0, _NCH // 2, pair_body, 0)


_PBLK = 16384


def _pad_body(tT_ref, eye_ref, out_ref):
    x = tT_ref[...]
    xT = jax.lax.dot_general(
        x, eye_ref[...], (((0,), (0,)), ((), ())),
        preferred_element_type=jnp.float32)
    out_ref[:, :EMBED] = xT
    out_ref[:, EMBED:] = jnp.zeros((_PBLK, EMBED), jnp.float32)


def _pad_table(tableT):
    return pl.pallas_call(
        _pad_body,
        grid=((VOCAB + _PBLK - 1) // _PBLK,),
        in_specs=[
            pl.BlockSpec((EMBED, _PBLK), lambda i: (0, i)),
            pl.BlockSpec((EMBED, EMBED), lambda i: (0, 0)),
        ],
        out_specs=pl.BlockSpec((_PBLK, 2 * EMBED), lambda i: (i, 0)),
        out_shape=jax.ShapeDtypeStruct((VOCAB, 2 * EMBED), jnp.float32),
        compiler_params=pltpu.CompilerParams(
            dimension_semantics=("arbitrary",)),
    )(tableT, jnp.eye(EMBED, dtype=jnp.float32))


_BB = 1024
_NB = B // _BB
_TG = 10
_NG = T // _TG


def _lstm_body(*refs):
    emb_refs = refs[:_TG]
    w_ref, bi_ref, bh_ref, out_ref, h_ref, c_ref = refs[_TG:]
    g = pl.program_id(1)

    @pl.when(g == 0)
    def _():
        h_ref[...] = jnp.zeros_like(h_ref)
        c_ref[...] = jnp.zeros_like(c_ref)

    bias = bi_ref[...] + bh_ref[...]

    def step(k):
        xh = jnp.concatenate([emb_refs[k][...], h_ref[...]], axis=1)
        gates = jnp.dot(
            xh, w_ref[...], preferred_element_type=jnp.float32) + bias
        i = jax.nn.sigmoid(gates[:, 0 * HID:1 * HID])
        f = jax.nn.sigmoid(gates[:, 1 * HID:2 * HID])
        gg = jnp.tanh(gates[:, 2 * HID:3 * HID])
        o = jax.nn.sigmoid(gates[:, 3 * HID:4 * HID])
        c = f * c_ref[...] + i * gg
        h = o * jnp.tanh(c)
        c_ref[...] = c
        h_ref[...] = h
        out_ref[k, :, :] = h

    for k in range(_TG):
        step(k)


def _emb_spec(k):
    return pl.BlockSpec(
        (_BB, 2 * EMBED),
        lambda b, g, k=k: ((g * _TG + k) * _NB + b, 0))


def _lstm(emb, w, bi, bh, *, interpret=False):
    return pl.pallas_call(
        _lstm_body,
        grid=(_NB, _NG),
        in_specs=[_emb_spec(k) for k in range(_TG)] + [
            pl.BlockSpec((2 * EMBED + HID, 4 * HID), lambda b, g: (0, 0)),
            pl.BlockSpec((1, 4 * HID), lambda b, g: (0, 0)),
            pl.BlockSpec((1, 4 * HID), lambda b, g: (0, 0)),
        ],
        out_specs=[
            pl.BlockSpec((_TG, _BB, HID), lambda b, g: (g, b, 0)),
            pl.BlockSpec((_BB, HID), lambda b, g: (b, 0)),
            pl.BlockSpec((_BB, HID), lambda b, g: (b, 0)),
        ],
        out_shape=[
            jax.ShapeDtypeStruct((T, B, HID), jnp.float32),
            jax.ShapeDtypeStruct((B, HID), jnp.float32),
            jax.ShapeDtypeStruct((B, HID), jnp.float32),
        ],
        compiler_params=pltpu.CompilerParams(
            dimension_semantics=("arbitrary", "arbitrary")),
        interpret=interpret,
    )(*([emb] * _TG), w, bi, bh)


def kernel(x, table, W_ih, W_hh, b_ih, b_hh):
    idx = x.astype(jnp.int32).T.reshape(_N)
    table128 = _pad_table(table.T)
    emb = _make_gather()(table128, idx)
    w = jnp.concatenate([
        W_ih.T,
        jnp.zeros((EMBED, 4 * HID), jnp.float32),
        W_hh.T,
    ], axis=0)
    out3, hT, cT = _lstm(emb, w,
                         b_ih.reshape(1, -1), b_hh.reshape(1, -1))
    return jnp.transpose(out3, (1, 0, 2)), hT, cT
